# baseline (device time: 113673 ns/iter reference)
import os

import jax
import jax.numpy as jnp
from jax import lax
from jax.experimental import pallas as pl
from jax.experimental.pallas import tpu as pltpu

N_DEV = 8
N_SUB = 4
N_HALF = 2
_SKIP_COMPUTE = os.environ.get("ABLATE_COMPUTE", "0") == "1"
_SKIP_COMM = os.environ.get("ABLATE_COMM", "0") == "1"


def kernel(x, w_mat, scale_x, scale_w):
    m_per, k = x.shape
    _, n_per = w_mat.shape
    m_sub = m_per // N_SUB
    m_half = m_sub // N_HALF

    s = (scale_x.reshape(-1)[:1] * scale_w.reshape(-1)[:1]).astype(jnp.float32)

    def body(x_ref, w_ref, s_ref, out_ref, *scratch):
        bufs = scratch[0:N_SUB]
        sends = scratch[N_SUB:2 * N_SUB]
        recvs = scratch[2 * N_SUB:3 * N_SUB]
        w8_ref = scratch[3 * N_SUB]

        my = lax.axis_index("i")

        def perm(r):
            return jnp.where(r < 4, r, 11 - r)

        my_r = perm(my)
        left = perm(lax.rem(my_r + N_DEV - 1, N_DEV))
        right = perm(lax.rem(my_r + 1, N_DEV))

        target = {0: right, 1: right, 2: left, 3: left}

        barrier_sem = pltpu.get_barrier_semaphore()
        for nbr in (left, right):
            pl.semaphore_signal(
                barrier_sem, inc=1,
                device_id=(nbr,), device_id_type=pl.DeviceIdType.MESH,
            )
        pl.semaphore_wait(barrier_sem, 2)

        def hop(si, h, half):
            rows = slice(half * m_half, (half + 1) * m_half)
            return pltpu.make_async_remote_copy(
                src_ref=bufs[si].at[h, rows],
                dst_ref=bufs[si].at[h + 1, rows],
                send_sem=sends[si].at[h, half],
                recv_sem=recvs[si].at[h, half],
                device_id=(target[si],),
                device_id_type=pl.DeviceIdType.MESH,
            )

        def store(si, h, origin):
            if _SKIP_COMPUTE:
                return
            acc = lax.dot_general(
                bufs[si][h], w8_ref[...],
                (((1,), (0,)), ((), ())),
                preferred_element_type=jnp.float32,
            )
            out_ref[pl.ds(origin * m_per + si * m_sub, m_sub), :] = acc * s_ref[0]

        for si in range(N_SUB):
            bufs[si][0] = x_ref[si * m_sub:(si + 1) * m_sub, :].astype(
                jnp.float8_e4m3fn)
        if not _SKIP_COMM:
            for half in range(N_HALF):
                for si in range(N_SUB):
                    hop(si, 0, half).start()
        w8_ref[...] = w_ref[...].astype(jnp.float8_e5m2)
        for si in range(N_SUB):
            store(si, 0, my)

        for h in range(N_DEV - 1):
            cw_origin = perm(lax.rem(my_r + N_DEV - 1 - h, N_DEV))
            ccw_origin = perm(lax.rem(my_r + 1 + h, N_DEV))
            for pair in ((0, 2), (1, 3)):
                if not _SKIP_COMM:
                    for half in range(N_HALF):
                        for si in pair:
                            hop(si, h, half).wait_recv()
                            if h < N_DEV - 2:
                                hop(si, h + 1, half).start()
                store(pair[0], h + 1, cw_origin)
                store(pair[1], h + 1, ccw_origin)

        if not _SKIP_COMM:
            for si in range(N_SUB):
                for h in range(N_DEV - 1):
                    for half in range(N_HALF):
                        hop(si, h, half).wait_send()

    comm = pltpu.VMEM((N_DEV, m_sub, k), jnp.float8_e4m3fn)
    sems = pltpu.SemaphoreType.DMA((N_DEV - 1, N_HALF))
    return pl.pallas_call(
        body,
        out_shape=jax.ShapeDtypeStruct((N_DEV * m_per, n_per), jnp.float32),
        in_specs=[
            pl.BlockSpec(memory_space=pltpu.VMEM),
            pl.BlockSpec(memory_space=pltpu.VMEM),
            pl.BlockSpec(memory_space=pltpu.SMEM),
        ],
        out_specs=pl.BlockSpec(memory_space=pltpu.VMEM),
        scratch_shapes=[comm] * N_SUB + [sems] * (2 * N_SUB) + [
            pltpu.VMEM((k, n_per), jnp.float8_e5m2),
        ],
        compiler_params=pltpu.CompilerParams(
            collective_id=0, vmem_limit_bytes=100 * 1024 * 1024),
    )(x, w_mat, s)


# device time: 72262 ns/iter; 1.5731x vs baseline; 1.5731x over previous
import os

import jax
import jax.numpy as jnp
from jax import lax
from jax.experimental import pallas as pl
from jax.experimental.pallas import tpu as pltpu

N_DEV = 8
N_SUB = 4
N_HALF = 2
_SKIP_COMPUTE = os.environ.get("ABLATE_COMPUTE", "0") == "1"
_SKIP_COMM = os.environ.get("ABLATE_COMM", "0") == "1"
_HALVES = 1 if os.environ.get("ABLATE_HALF", "0") == "1" else N_HALF


def kernel(x, w_mat, scale_x, scale_w):
    m_per, k = x.shape
    _, n_per = w_mat.shape
    m_sub = m_per // N_SUB
    m_half = m_sub // N_HALF

    s = (scale_x.reshape(-1)[:1] * scale_w.reshape(-1)[:1]).astype(jnp.float32)

    def body(x_ref, w_ref, s_ref, out_ref, *scratch):
        bufs = scratch[0:N_SUB]
        sends = scratch[N_SUB:2 * N_SUB]
        recvs = scratch[2 * N_SUB:3 * N_SUB]
        w8_ref = scratch[3 * N_SUB]

        my = lax.axis_index("i")

        def perm(r):
            return jnp.where(r < 4, r, 11 - r)

        my_r = perm(my)
        left = perm(lax.rem(my_r + N_DEV - 1, N_DEV))
        right = perm(lax.rem(my_r + 1, N_DEV))

        target = {0: right, 1: right, 2: left, 3: left}

        barrier_sem = pltpu.get_barrier_semaphore()
        for nbr in (left, right):
            pl.semaphore_signal(
                barrier_sem, inc=1,
                device_id=(nbr,), device_id_type=pl.DeviceIdType.MESH,
            )
        pl.semaphore_wait(barrier_sem, 2)

        def hop(si, h, half):
            rows = slice(half * m_half, (half + 1) * m_half)
            return pltpu.make_async_remote_copy(
                src_ref=bufs[si].at[h, rows],
                dst_ref=bufs[si].at[h + 1, rows],
                send_sem=sends[si].at[h, half],
                recv_sem=recvs[si].at[h, half],
                device_id=(target[si],),
                device_id_type=pl.DeviceIdType.MESH,
            )

        def store(si, h, origin):
            if _SKIP_COMPUTE:
                return
            acc = lax.dot_general(
                bufs[si][h], w8_ref[...],
                (((1,), (0,)), ((), ())),
                preferred_element_type=jnp.float32,
            )
            out_ref[pl.ds(origin * m_per + si * m_sub, m_sub), :] = acc * s_ref[0]

        for si in range(N_SUB):
            bufs[si][0] = x_ref[si * m_sub:(si + 1) * m_sub, :].astype(
                jnp.float8_e4m3fn)
        if not _SKIP_COMM:
            for half in range(_HALVES):
                for si in range(N_SUB):
                    hop(si, 0, half).start()
        w8_ref[...] = w_ref[...].astype(jnp.float8_e5m2)
        for si in range(N_SUB):
            store(si, 0, my)

        for h in range(N_DEV - 1):
            cw_origin = perm(lax.rem(my_r + N_DEV - 1 - h, N_DEV))
            ccw_origin = perm(lax.rem(my_r + 1 + h, N_DEV))
            for pair in ((0, 2), (1, 3)):
                if not _SKIP_COMM:
                    for half in range(_HALVES):
                        for si in pair:
                            hop(si, h, half).wait_recv()
                            if h < N_DEV - 2:
                                hop(si, h + 1, half).start()
                store(pair[0], h + 1, cw_origin)
                store(pair[1], h + 1, ccw_origin)

        if not _SKIP_COMM:
            for si in range(N_SUB):
                for h in range(N_DEV - 1):
                    for half in range(_HALVES):
                        hop(si, h, half).wait_send()

    comm = pltpu.VMEM((N_DEV, m_sub, k), jnp.float8_e4m3fn)
    sems = pltpu.SemaphoreType.DMA((N_DEV - 1, N_HALF))
    return pl.pallas_call(
        body,
        out_shape=jax.ShapeDtypeStruct((N_DEV * m_per, n_per), jnp.float32),
        in_specs=[
            pl.BlockSpec(memory_space=pltpu.VMEM),
            pl.BlockSpec(memory_space=pltpu.VMEM),
            pl.BlockSpec(memory_space=pltpu.SMEM),
        ],
        out_specs=pl.BlockSpec(memory_space=pltpu.VMEM),
        scratch_shapes=[comm] * N_SUB + [sems] * (2 * N_SUB) + [
            pltpu.VMEM((k, n_per), jnp.float8_e5m2),
        ],
        compiler_params=pltpu.CompilerParams(
            collective_id=0, vmem_limit_bytes=100 * 1024 * 1024),
    )(x, w_mat, s)
